# baseline (device time: 209207 ns/iter reference)
import jax
import jax.numpy as jnp
from jax import lax
from jax.experimental import pallas as pl
from jax.experimental.pallas import tpu as pltpu

N_DEV = 16
E_LOCAL = 4


def kernel(x, router_W, route_idx, expert_W):
    n_tok, d_model = x.shape
    n_exp = router_W.shape[1]
    d_out = expert_W.shape[2]

    def body(x_ref, rw_ref, idx_ref, ew_ref, out_ref, comm_ref,
             send_sems, recv_sems):
        my_pos = lax.axis_index("i")
        left = lax.rem(my_pos - 1 + N_DEV, N_DEV)
        right = lax.rem(my_pos + 1, N_DEV)

        barrier_sem = pltpu.get_barrier_semaphore()
        for nbr in (left, right):
            pl.semaphore_signal(
                barrier_sem, inc=1,
                device_id=(nbr,), device_id_type=pl.DeviceIdType.MESH,
            )
        pl.semaphore_wait(barrier_sem, 2)

        xv = x_ref[:, :]
        scores = jnp.dot(xv, rw_ref[:, :],
                         preferred_element_type=jnp.float32,
                         precision=lax.Precision.HIGHEST)
        probs = jnp.exp(scores - jnp.max(scores, axis=1, keepdims=True))
        probs = probs / jnp.sum(probs, axis=1, keepdims=True)

        iota_e = lax.broadcasted_iota(jnp.int32, (n_tok, n_exp), 1)
        sel0 = idx_ref[:, 0:1] == iota_e
        sel1 = idx_ref[:, 1:2] == iota_e
        g0 = jnp.sum(jnp.where(sel0, probs, 0.0), axis=1, keepdims=True)
        g1 = jnp.sum(jnp.where(sel1, probs, 0.0), axis=1, keepdims=True)
        wts = jnp.where(jnp.logical_or(sel0, sel1), probs, 0.0) / (g0 + g1)

        partial = jnp.zeros((n_tok, d_out), jnp.float32)
        for el in range(E_LOCAL):
            w_el = jnp.sum(
                jnp.where(iota_e == my_pos * E_LOCAL + el, wts, 0.0),
                axis=1, keepdims=True)
            partial = partial + jnp.dot(
                xv * w_el, ew_ref[el],
                preferred_element_type=jnp.float32)

        out_ref[:, :] = partial
        comm_ref[0, :, :] = partial

        for h in range(N_DEV - 1):
            rdma = pltpu.make_async_remote_copy(
                src_ref=comm_ref.at[h],
                dst_ref=comm_ref.at[h + 1],
                send_sem=send_sems.at[h],
                recv_sem=recv_sems.at[h],
                device_id=(right,),
                device_id_type=pl.DeviceIdType.MESH,
            )
            rdma.start()
            rdma.wait()
            out_ref[:, :] = out_ref[:, :] + comm_ref[h + 1, :, :]

    return pl.pallas_call(
        body,
        out_shape=jax.ShapeDtypeStruct((n_tok, d_out), jnp.float32),
        in_specs=[
            pl.BlockSpec(memory_space=pltpu.VMEM),
            pl.BlockSpec(memory_space=pltpu.VMEM),
            pl.BlockSpec(memory_space=pltpu.VMEM),
            pl.BlockSpec(memory_space=pltpu.VMEM),
        ],
        out_specs=pl.BlockSpec(memory_space=pltpu.VMEM),
        scratch_shapes=[
            pltpu.VMEM((N_DEV, n_tok, d_out), jnp.float32),
            pltpu.SemaphoreType.DMA((N_DEV - 1,)),
            pltpu.SemaphoreType.DMA((N_DEV - 1,)),
        ],
        compiler_params=pltpu.CompilerParams(collective_id=0),
    )(x, router_W, route_idx, expert_W)


# device time: 87178 ns/iter; 2.3998x vs baseline; 2.3998x over previous
import jax
import jax.numpy as jnp
from jax import lax
from jax.experimental import pallas as pl
from jax.experimental.pallas import tpu as pltpu

N_DEV = 16
E_LOCAL = 4


def kernel(x, router_W, route_idx, expert_W):
    n_tok, d_model = x.shape
    n_exp = router_W.shape[1]
    d_out = expert_W.shape[2]
    rows = n_tok // N_DEV

    def body(x_ref, rw_ref, idx_ref, ew_ref, out_ref, rs_buf,
             send_sems, recv_sems):
        my_pos = lax.axis_index("i")
        left = lax.rem(my_pos - 1 + N_DEV, N_DEV)
        right = lax.rem(my_pos + 1, N_DEV)

        barrier_sem = pltpu.get_barrier_semaphore()
        for nbr in (left, right):
            pl.semaphore_signal(
                barrier_sem, inc=1,
                device_id=(nbr,), device_id_type=pl.DeviceIdType.MESH,
            )
        pl.semaphore_wait(barrier_sem, 2)

        xv = x_ref[:, :]
        scores = jnp.dot(xv, rw_ref[:, :],
                         preferred_element_type=jnp.float32,
                         precision=lax.Precision.HIGHEST)
        probs = jnp.exp(scores - jnp.max(scores, axis=1, keepdims=True))
        probs = probs / jnp.sum(probs, axis=1, keepdims=True)

        iota_e = lax.broadcasted_iota(jnp.int32, (n_tok, n_exp), 1)
        sel0 = idx_ref[:, 0:1] == iota_e
        sel1 = idx_ref[:, 1:2] == iota_e
        g0 = jnp.sum(jnp.where(sel0, probs, 0.0), axis=1, keepdims=True)
        g1 = jnp.sum(jnp.where(sel1, probs, 0.0), axis=1, keepdims=True)
        wts = jnp.where(jnp.logical_or(sel0, sel1), probs, 0.0) / (g0 + g1)

        partial = jnp.zeros((n_tok, d_out), jnp.float32)
        for el in range(E_LOCAL):
            w_el = jnp.sum(
                jnp.where(iota_e == my_pos * E_LOCAL + el, wts, 0.0),
                axis=1, keepdims=True)
            partial = partial + jnp.dot(
                xv * w_el, ew_ref[el],
                preferred_element_type=jnp.float32)

        out_ref[:, :] = partial

        def chunk(ref, c):
            return ref.at[pl.ds(lax.rem(c + 2 * N_DEV, N_DEV) * rows, rows), :]

        for h in range(N_DEV - 1):
            rdma = pltpu.make_async_remote_copy(
                src_ref=chunk(out_ref, my_pos - h),
                dst_ref=rs_buf.at[h],
                send_sem=send_sems.at[h],
                recv_sem=recv_sems.at[h],
                device_id=(right,),
                device_id_type=pl.DeviceIdType.MESH,
            )
            rdma.start()
            rdma.wait()
            rc = lax.rem(my_pos - h - 1 + 2 * N_DEV, N_DEV) * rows
            out_ref[pl.ds(rc, rows), :] = (
                out_ref[pl.ds(rc, rows), :] + rs_buf[h, :, :])

        for g in range(N_DEV - 1):
            rdma = pltpu.make_async_remote_copy(
                src_ref=chunk(out_ref, my_pos + 1 - g),
                dst_ref=chunk(out_ref, my_pos + 1 - g),
                send_sem=send_sems.at[N_DEV - 1 + g],
                recv_sem=recv_sems.at[N_DEV - 1 + g],
                device_id=(right,),
                device_id_type=pl.DeviceIdType.MESH,
            )
            rdma.start()
            rdma.wait()

    return pl.pallas_call(
        body,
        out_shape=jax.ShapeDtypeStruct((n_tok, d_out), jnp.float32),
        in_specs=[
            pl.BlockSpec(memory_space=pltpu.VMEM),
            pl.BlockSpec(memory_space=pltpu.VMEM),
            pl.BlockSpec(memory_space=pltpu.VMEM),
            pl.BlockSpec(memory_space=pltpu.VMEM),
        ],
        out_specs=pl.BlockSpec(memory_space=pltpu.VMEM),
        scratch_shapes=[
            pltpu.VMEM((N_DEV - 1, rows, d_out), jnp.float32),
            pltpu.SemaphoreType.DMA((2 * (N_DEV - 1),)),
            pltpu.SemaphoreType.DMA((2 * (N_DEV - 1),)),
        ],
        compiler_params=pltpu.CompilerParams(collective_id=0),
    )(x, router_W, route_idx, expert_W)


# device time: 47358 ns/iter; 4.4176x vs baseline; 1.8408x over previous
import jax
import jax.numpy as jnp
from jax import lax
from jax.experimental import pallas as pl
from jax.experimental.pallas import tpu as pltpu

N_DEV = 16
E_LOCAL = 4


def kernel(x, router_W, route_idx, expert_W):
    n_tok, d_model = x.shape
    n_exp = router_W.shape[1]
    d_out = expert_W.shape[2]

    def body(x_ref, rw_ref, idx_ref, ew_ref, out_ref,
             rs_x, rs_y, rs_z1, rs_z2, send_sems, recv_sems):
        my_pos = lax.axis_index("i")
        z = lax.div(my_pos, 4)
        p = lax.rem(my_pos, 4)
        by = lax.div(p, 2)
        bx = lax.rem(p + by, 2)
        bz1 = lax.rem(z, 2)
        bz2 = lax.div(z, 2)
        px = 4 * z + (p + 1 - 2 * lax.rem(p, 2))
        py = 4 * z + (3 - p)
        pz1 = 4 * (z + 1 - 2 * bz1) + p
        pz2 = 4 * (z + 2 - 4 * bz2) + p

        barrier_sem = pltpu.get_barrier_semaphore()
        for nbr in (px, py, pz1, pz2):
            pl.semaphore_signal(
                barrier_sem, inc=1,
                device_id=(nbr,), device_id_type=pl.DeviceIdType.MESH,
            )
        pl.semaphore_wait(barrier_sem, 4)

        xv = x_ref[:, :]
        scores = jnp.dot(xv, rw_ref[:, :],
                         preferred_element_type=jnp.float32,
                         precision=lax.Precision.HIGHEST)
        probs = jnp.exp(scores - jnp.max(scores, axis=1, keepdims=True))
        probs = probs / jnp.sum(probs, axis=1, keepdims=True)

        iota_e = lax.broadcasted_iota(jnp.int32, (n_tok, n_exp), 1)
        sel0 = idx_ref[:, 0:1] == iota_e
        sel1 = idx_ref[:, 1:2] == iota_e
        g0 = jnp.sum(jnp.where(sel0, probs, 0.0), axis=1, keepdims=True)
        g1 = jnp.sum(jnp.where(sel1, probs, 0.0), axis=1, keepdims=True)
        wts = jnp.where(jnp.logical_or(sel0, sel1), probs, 0.0) / (g0 + g1)

        partial = jnp.zeros((n_tok, d_out), jnp.float32)
        for el in range(E_LOCAL):
            w_el = jnp.sum(
                jnp.where(iota_e == my_pos * E_LOCAL + el, wts, 0.0),
                axis=1, keepdims=True)
            partial = partial + jnp.dot(
                xv * w_el, ew_ref[el],
                preferred_element_type=jnp.float32)

        out_ref[:, :] = partial

        off1 = bx * 256
        off2 = off1 + by * 128
        off3 = off2 + bz1 * 64
        off4 = off3 + bz2 * 32

        rs_steps = [
            (px, (1 - bx) * 256, off1, 256, rs_x),
            (py, off1 + (1 - by) * 128, off2, 128, rs_y),
            (pz1, off2 + (1 - bz1) * 64, off3, 64, rs_z1),
            (pz2, off3 + (1 - bz2) * 32, off4, 32, rs_z2),
        ]
        for k, (tgt, s_off, k_off, nrows, rbuf) in enumerate(rs_steps):
            rdma = pltpu.make_async_remote_copy(
                src_ref=out_ref.at[pl.ds(s_off, nrows), :],
                dst_ref=rbuf,
                send_sem=send_sems.at[k],
                recv_sem=recv_sems.at[k],
                device_id=(tgt,),
                device_id_type=pl.DeviceIdType.MESH,
            )
            rdma.start()
            rdma.wait()
            out_ref[pl.ds(k_off, nrows), :] = (
                out_ref[pl.ds(k_off, nrows), :] + rbuf[:, :])

        ag_steps = [
            (pz2, off4, 32),
            (pz1, off3, 64),
            (py, off2, 128),
            (px, off1, 256),
        ]
        for k, (tgt, s_off, nrows) in enumerate(ag_steps):
            rdma = pltpu.make_async_remote_copy(
                src_ref=out_ref.at[pl.ds(s_off, nrows), :],
                dst_ref=out_ref.at[pl.ds(s_off, nrows), :],
                send_sem=send_sems.at[4 + k],
                recv_sem=recv_sems.at[4 + k],
                device_id=(tgt,),
                device_id_type=pl.DeviceIdType.MESH,
            )
            rdma.start()
            rdma.wait()

    return pl.pallas_call(
        body,
        out_shape=jax.ShapeDtypeStruct((n_tok, d_out), jnp.float32),
        in_specs=[
            pl.BlockSpec(memory_space=pltpu.VMEM),
            pl.BlockSpec(memory_space=pltpu.VMEM),
            pl.BlockSpec(memory_space=pltpu.VMEM),
            pl.BlockSpec(memory_space=pltpu.VMEM),
        ],
        out_specs=pl.BlockSpec(memory_space=pltpu.VMEM),
        scratch_shapes=[
            pltpu.VMEM((256, d_out), jnp.float32),
            pltpu.VMEM((128, d_out), jnp.float32),
            pltpu.VMEM((64, d_out), jnp.float32),
            pltpu.VMEM((32, d_out), jnp.float32),
            pltpu.SemaphoreType.DMA((8,)),
            pltpu.SemaphoreType.DMA((8,)),
        ],
        compiler_params=pltpu.CompilerParams(collective_id=0),
    )(x, router_W, route_idx, expert_W)


# device time: 28457 ns/iter; 7.3517x vs baseline; 1.6642x over previous
import jax
import jax.numpy as jnp
from jax import lax
from jax.experimental import pallas as pl
from jax.experimental.pallas import tpu as pltpu

N_DEV = 16
E_LOCAL = 4


def kernel(x, router_W, route_idx, expert_W):
    n_tok, d_model = x.shape
    n_exp = router_W.shape[1]
    d_out = expert_W.shape[2]

    def body(x_ref, rw_ref, idx_ref, ew_ref, out_ref,
             snd, rcv, send_sems, recv_sems):
        my_pos = lax.axis_index("i")
        z = lax.div(my_pos, 4)
        p = lax.rem(my_pos, 4)
        by = lax.div(p, 2)
        bx = lax.rem(p + by, 2)
        bz1 = lax.rem(z, 2)
        bz2 = lax.div(z, 2)
        px = 4 * z + (p + 1 - 2 * lax.rem(p, 2))
        py = 4 * z + (3 - p)
        pz1 = 4 * (z + 1 - 2 * bz1) + p
        pz2 = 4 * (z + 2 - 4 * bz2) + p

        barrier_sem = pltpu.get_barrier_semaphore()
        for nbr in (px, py, pz1, pz2):
            pl.semaphore_signal(
                barrier_sem, inc=1,
                device_id=(nbr,), device_id_type=pl.DeviceIdType.MESH,
            )

        xv = x_ref[:, :]
        scores = jnp.dot(xv, rw_ref[:, :],
                         preferred_element_type=jnp.float32,
                         precision=lax.Precision.HIGHEST)
        probs = jnp.exp(scores - jnp.max(scores, axis=1, keepdims=True))
        probs = probs / jnp.sum(probs, axis=1, keepdims=True)

        iota_e = lax.broadcasted_iota(jnp.int32, (n_tok, n_exp), 1)
        sel0 = idx_ref[:, 0:1] == iota_e
        sel1 = idx_ref[:, 1:2] == iota_e
        g0 = jnp.sum(jnp.where(sel0, probs, 0.0), axis=1, keepdims=True)
        g1 = jnp.sum(jnp.where(sel1, probs, 0.0), axis=1, keepdims=True)
        wts = jnp.where(jnp.logical_or(sel0, sel1), probs, 0.0) / (g0 + g1)
        w_el = [
            jnp.sum(jnp.where(iota_e == my_pos * E_LOCAL + el, wts, 0.0),
                    axis=1, keepdims=True)
            for el in range(E_LOCAL)
        ]

        a1 = bx * 128
        a2 = a1 + by * 64
        b1 = 256 + by * 128
        b2 = b1 + bx * 64

        def start_send(sem_idx, tgt, nrows=64):
            rdma = pltpu.make_async_remote_copy(
                src_ref=snd.at[sem_idx, pl.ds(0, nrows), :],
                dst_ref=rcv.at[sem_idx, pl.ds(0, nrows), :],
                send_sem=send_sems.at[sem_idx],
                recv_sem=recv_sems.at[sem_idx],
                device_id=(tgt,), device_id_type=pl.DeviceIdType.MESH,
            )
            rdma.start()
            return rdma

        def gemm_block(r0, stage=None):
            blk = jnp.zeros((128, d_out), jnp.float32)
            for el in range(E_LOCAL):
                blk = blk + jnp.dot(
                    xv[r0:r0 + 128] * w_el[el][r0:r0 + 128],
                    ew_ref[el], preferred_element_type=jnp.float32)
            out_ref[r0:r0 + 128, :] = blk
            if stage is not None:
                snd[stage, :, :] = blk.astype(jnp.bfloat16)

        def branch(bit, r_true, r_false, stage=None):
            pl.when(bit == 1)(lambda: gemm_block(r_true, stage))
            pl.when(bit == 0)(lambda: gemm_block(r_false, stage))

        branch(bx, 0, 128, stage=0)
        pl.semaphore_wait(barrier_sem, 4)
        rdma_a1 = start_send(0, px, 128)
        branch(by, 256, 384, stage=1)
        rdma_b1 = start_send(1, py, 128)
        branch(bx, 128, 0)
        branch(by, 384, 256)

        rdma_a1.wait()
        fwd = a1 + (1 - by) * 64
        sum_a = (out_ref[pl.ds(fwd, 64), :]
                 + rcv[0, pl.ds((1 - by) * 64, 64), :].astype(jnp.float32))
        snd[2, :64, :] = sum_a.astype(jnp.bfloat16)
        rdma_a2 = start_send(2, py)
        out_ref[pl.ds(fwd, 64), :] = sum_a
        out_ref[pl.ds(a2, 64), :] = (
            out_ref[pl.ds(a2, 64), :]
            + rcv[0, pl.ds(by * 64, 64), :].astype(jnp.float32))

        rdma_b1.wait()
        fwd = b1 + (1 - bx) * 64
        sum_b = (out_ref[pl.ds(fwd, 64), :]
                 + rcv[1, pl.ds((1 - bx) * 64, 64), :].astype(jnp.float32))
        snd[3, :64, :] = sum_b.astype(jnp.bfloat16)
        rdma_b2 = start_send(3, px)
        out_ref[pl.ds(fwd, 64), :] = sum_b
        out_ref[pl.ds(b2, 64), :] = (
            out_ref[pl.ds(b2, 64), :]
            + rcv[1, pl.ds(bx * 64, 64), :].astype(jnp.float32))

        def fused_step(in_sem, out_sem, tgt, off):
            s = (out_ref[pl.ds(off, 64), :]
                 + rcv[in_sem, :64, :].astype(jnp.float32))
            snd[out_sem, :64, :] = s.astype(jnp.bfloat16)
            rdma = start_send(out_sem, tgt)
            out_ref[pl.ds(off, 64), :] = s
            return rdma

        rdma_a2.wait()
        rdma_az1 = fused_step(2, 4, pz1, a2)
        rdma_b2.wait()
        rdma_bz1 = fused_step(3, 5, pz1, b2)

        rdma_az1.wait()
        rdma_az2 = fused_step(4, 6, pz2, a2)
        rdma_bz1.wait()
        rdma_bz2 = fused_step(5, 7, pz2, b2)

        rdma_az2.wait()
        ag_ay = fused_step(6, 8, py, a2)
        rdma_bz2.wait()
        ag_bx = fused_step(7, 9, px, b2)

        ag_ay.wait()
        snd[10, pl.ds(by * 64, 64), :] = (
            out_ref[pl.ds(a2, 64), :].astype(jnp.bfloat16))
        snd[10, pl.ds((1 - by) * 64, 64), :] = rcv[8, :64, :]
        ag_ax = start_send(10, px, 128)
        out_ref[pl.ds(a1 + (1 - by) * 64, 64), :] = (
            rcv[8, :64, :].astype(jnp.float32))

        ag_bx.wait()
        snd[11, pl.ds(bx * 64, 64), :] = (
            out_ref[pl.ds(b2, 64), :].astype(jnp.bfloat16))
        snd[11, pl.ds((1 - bx) * 64, 64), :] = rcv[9, :64, :]
        ag_by = start_send(11, py, 128)
        out_ref[pl.ds(b1 + (1 - bx) * 64, 64), :] = (
            rcv[9, :64, :].astype(jnp.float32))

        ag_ax.wait()
        out_ref[pl.ds((1 - bx) * 128, 128), :] = (
            rcv[10, :, :].astype(jnp.float32))
        ag_by.wait()
        out_ref[pl.ds(256 + (1 - by) * 128, 128), :] = (
            rcv[11, :, :].astype(jnp.float32))

    return pl.pallas_call(
        body,
        out_shape=jax.ShapeDtypeStruct((n_tok, d_out), jnp.float32),
        in_specs=[
            pl.BlockSpec(memory_space=pltpu.VMEM),
            pl.BlockSpec(memory_space=pltpu.VMEM),
            pl.BlockSpec(memory_space=pltpu.VMEM),
            pl.BlockSpec(memory_space=pltpu.VMEM),
        ],
        out_specs=pl.BlockSpec(memory_space=pltpu.VMEM),
        scratch_shapes=[
            pltpu.VMEM((12, 128, d_out), jnp.bfloat16),
            pltpu.VMEM((12, 128, d_out), jnp.bfloat16),
            pltpu.SemaphoreType.DMA((12,)),
            pltpu.SemaphoreType.DMA((12,)),
        ],
        compiler_params=pltpu.CompilerParams(collective_id=0),
    )(x, router_W, route_idx, expert_W)
